# baseline (speedup 1.0000x reference)
import functools

import jax
import jax.numpy as jnp
from jax import lax
from jax.experimental import pallas as pl
from jax.experimental.pallas import tpu as pltpu
from jax.experimental.pallas import tpu_sc as plsc

_NC = 2
_NS = 16
_NW = _NC * _NS
_LANES = 16
_RPE = 4


def _pool_stage(x2, table, Wt, b, inv_seq, DP):
  V, E = table.shape
  C = Wt.shape[0]
  B2, H = x2.shape
  B = B2 // 2
  KC = E // _LANES
  per_w = B // _NW
  NJ = 2 * per_w
  NCH = NJ // 8
  NSLOT = 8
  DIST = 6

  mesh = plsc.VectorSubcoreMesh(core_axis_name="c", subcore_axis_name="s")

  @functools.partial(
      pl.kernel,
      out_type=jax.ShapeDtypeStruct((B, DP), jnp.float32),
      mesh=mesh,
      scratch_types=[
          pltpu.VMEM((3, 8, H), jnp.int32),
          pltpu.VMEM((NSLOT, H, E), jnp.float32),
          pltpu.VMEM((2, 8, DP), jnp.float32),
          pltpu.VMEM((C, E), jnp.float32),
          pltpu.VMEM((_LANES,), jnp.float32),
          [pltpu.SemaphoreType.DMA] * NSLOT,
          [pltpu.SemaphoreType.DMA] * 3,
          pltpu.SemaphoreType.DMA,
      ],
  )
  def body(
      x2_hbm, tab_hbm, wt_hbm, b_hbm, out_hbm,
      ich_v, rows_v, out_v, w_v, b_v, sems, isems, sem_out,
  ):
    wid = lax.axis_index("s") * _NC + lax.axis_index("c")
    pltpu.sync_copy(wt_hbm, w_v)
    pltpu.sync_copy(b_hbm, b_v)
    bvals = b_v[...]
    wvec = [
        [w_v[c, pl.ds(k * _LANES, _LANES)] for k in range(KC)]
        for c in range(C)
    ]

    def fire_idx(ch, slot):
      pltpu.async_copy(
          x2_hbm.at[pl.ds(pl.multiple_of(wid * NJ + ch * 8, 8), 8)],
          ich_v.at[slot],
          isems[slot],
      )

    def drain_idx(slot):
      pltpu.make_async_copy(
          x2_hbm.at[pl.ds(0, 8)], ich_v.at[slot], isems[slot]
      ).wait()

    def fire(idx_ref, slot):
      pltpu.async_copy(tab_hbm.at[idx_ref], rows_v.at[slot], sems[slot])

    def drain(slot):
      pltpu.make_async_copy(
          tab_hbm.at[ich_v.at[0, 0]], rows_v.at[slot], sems[slot]
      ).wait()

    def drain_out():
      pltpu.make_async_copy(
          out_v.at[0],
          out_hbm.at[pl.ds(pl.multiple_of(wid * per_w, 8), 8)],
          sem_out,
      ).wait()

    def partial_reduce(slot, acc):
      def g_body(g, acc):
        out = list(acc)
        for u in range(5):
          t = 5 * g + u
          for k in range(KC):
            out[k] = out[k] + rows_v[slot, t, pl.ds(k * _LANES, _LANES)]
        return tuple(out)
      return lax.fori_loop(0, H // 5, g_body, acc)

    def finalize_elem(e, acc):
      @pl.when((e % 8 == 0) & (e >= 16))
      def _():
        drain_out()
      buf = (e // 8) % 2
      row = e % 8
      lanes = lax.iota(jnp.int32, _LANES)
      logit_vec = jnp.zeros((_LANES,), jnp.float32)
      for c in range(C):
        m = acc[0] * wvec[c][0]
        for k in range(1, KC):
          m = m + acc[k] * wvec[c][k]
        for fold in (8, 4, 2, 1):
          perm = jnp.bitwise_xor(lanes, fold)
          m = m + m.at[perm].get(mode="promise_in_bounds")
        o = m * inv_seq + bvals[c]
        logit_vec = jnp.where(lanes == c, o, logit_vec)
      out_v[buf, row, :] = logit_vec
      @pl.when(e % 8 == 7)
      def _():
        pltpu.async_copy(
            out_v.at[buf],
            out_hbm.at[pl.ds(pl.multiple_of(wid * per_w + e - 7, 8), 8)],
            sem_out,
        )

    def switch3(sel, fn):
      lax.switch(sel, [lambda s=s: fn(s) for s in range(3)])

    def run(i, carry):
      j0 = 8 * i
      c_cur = i % 3
      c_next = (i + 1) % 3
      for half in range(4):
        acc = tuple(jnp.zeros((_LANES,), jnp.float32) for _ in range(KC))
        for q in range(2):
          d = 2 * half + q
          jcur = j0 + d
          if d == 0:
            @pl.when(i + 2 < NCH)
            def _():
              switch3((i + 2) % 3, lambda s: fire_idx(i + 2, s))
          if d == 2:
            @pl.when(i + 1 < NCH)
            def _():
              switch3(c_next, drain_idx)
          @pl.when(jcur + DIST < NJ)
          def _():
            gslot = (d + DIST) % NSLOT
            if d < 2:
              switch3(
                  c_cur,
                  lambda s, r=d + DIST, g=gslot: fire(ich_v.at[s, r], g),
              )
            else:
              switch3(
                  c_next,
                  lambda s, r=d - 2, g=gslot: fire(ich_v.at[s, r], g),
              )
          drain(d)
          acc = partial_reduce(d, acc)
        finalize_elem(j0 // 2 + half, acc)
      return carry

    fire_idx(0, 0)
    fire_idx(1, 1)
    drain_idx(0)
    for j in range(DIST):
      fire(ich_v.at[0, j], j)
    lax.fori_loop(0, NCH, run, 0)
    drain_out()
    drain_out()

  return body(x2, table, Wt, b)


def kernel(x, table, W, b):
  B, S = x.shape
  C = W.shape[1]
  DP = _LANES

  x2 = x.astype(jnp.int32).reshape(B * 2, S // 2)
  b_pad = jnp.concatenate([b, jnp.zeros((_LANES - C,), b.dtype)])
  out_pad = _pool_stage(x2, table, W.T, b_pad, 1.0 / S, DP)
  return out_pad[:, :C]


# speedup vs baseline: 1.0064x; 1.0064x over previous
import functools

import jax
import jax.numpy as jnp
from jax import lax
from jax.experimental import pallas as pl
from jax.experimental.pallas import tpu as pltpu
from jax.experimental.pallas import tpu_sc as plsc

_NC = 2
_NS = 16
_NW = _NC * _NS
_LANES = 16
_RPE = 4


def _pool_stage(x2, table, Wt, b, inv_seq, DP):
  V, E = table.shape
  C = Wt.shape[0]
  BR, H = x2.shape
  B = BR // _RPE
  KC = E // _LANES
  per_w = B // _NW
  NSLOT = 2 * _RPE
  DIST = NSLOT - 2

  mesh = plsc.VectorSubcoreMesh(core_axis_name="c", subcore_axis_name="s")

  @functools.partial(
      pl.kernel,
      out_type=jax.ShapeDtypeStruct((B, DP), jnp.float32),
      mesh=mesh,
      scratch_types=[
          pltpu.VMEM((_RPE * per_w, H), jnp.int32),
          pltpu.VMEM((NSLOT, H, E), jnp.float32),
          pltpu.VMEM((2, 8, DP), jnp.float32),
          pltpu.VMEM((C, E), jnp.float32),
          pltpu.VMEM((_LANES,), jnp.float32),
          [pltpu.SemaphoreType.DMA] * NSLOT,
          pltpu.SemaphoreType.DMA,
      ],
  )
  def body(
      x2_hbm, tab_hbm, wt_hbm, b_hbm, out_hbm,
      idx_v, rows_v, out_v, w_v, b_v, sems, sem_out,
  ):
    wid = lax.axis_index("s") * _NC + lax.axis_index("c")
    NJ = _RPE * per_w
    pltpu.sync_copy(
        x2_hbm.at[pl.ds(pl.multiple_of(wid * NJ, 8), NJ)],
        idx_v,
    )
    pltpu.sync_copy(wt_hbm, w_v)
    pltpu.sync_copy(b_hbm, b_v)
    bvals = b_v[...]
    wvec = [
        [w_v[c, pl.ds(k * _LANES, _LANES)] for k in range(KC)]
        for c in range(C)
    ]

    def fire(j, slot):
      pltpu.async_copy(tab_hbm.at[idx_v.at[j]], rows_v.at[slot], sems[slot])

    def drain(slot):
      pltpu.make_async_copy(
          tab_hbm.at[idx_v.at[0]], rows_v.at[slot], sems[slot]
      ).wait()

    def drain_out():
      pltpu.make_async_copy(
          out_v.at[0],
          out_hbm.at[pl.ds(pl.multiple_of(wid * per_w, 8), 8)],
          sem_out,
      ).wait()

    def partial_reduce(slot, acc):
      def g_body(g, acc):
        out = list(acc)
        for u in range(5):
          t = 5 * g + u
          for k in range(KC):
            out[k] = out[k] + rows_v[slot, t, pl.ds(k * _LANES, _LANES)]
        return tuple(out)
      return lax.fori_loop(0, H // 5, g_body, acc)

    def finalize_elem(e, acc):
      @pl.when((e % 8 == 0) & (e >= 16))
      def _():
        drain_out()
      buf = (e // 8) % 2
      row = e % 8
      lanes = lax.iota(jnp.int32, _LANES)
      logit_vec = jnp.zeros((_LANES,), jnp.float32)
      for c in range(C):
        m = acc[0] * wvec[c][0]
        for k in range(1, KC):
          m = m + acc[k] * wvec[c][k]
        for fold in (8, 4, 2, 1):
          perm = jnp.bitwise_xor(lanes, fold)
          m = m + m.at[perm].get(mode="promise_in_bounds")
        o = m * inv_seq + bvals[c]
        logit_vec = jnp.where(lanes == c, o, logit_vec)
      out_v[buf, row, :] = logit_vec
      @pl.when(e % 8 == 7)
      def _():
        pltpu.async_copy(
            out_v.at[buf],
            out_hbm.at[pl.ds(pl.multiple_of(wid * per_w + e - 7, 8), 8)],
            sem_out,
        )

    def run(j0, carry):
      for half in range(2):
        acc = tuple(jnp.zeros((_LANES,), jnp.float32) for _ in range(KC))
        for q in range(_RPE):
          d = _RPE * half + q
          jcur = j0 + d
          @pl.when(jcur + DIST < NJ)
          def _():
            fire(jcur + DIST, (d + DIST) % NSLOT)
          drain(d)
          acc = partial_reduce(d, acc)
        finalize_elem(j0 // _RPE + half, acc)
      return carry

    for j in range(DIST):
      fire(j, j)
    lax.fori_loop(0, NJ // NSLOT, lambda i, c: run(NSLOT * i, c), 0)
    drain_out()
    drain_out()

  return body(x2, table, Wt, b)


def kernel(x, table, W, b):
  B, S = x.shape
  C = W.shape[1]
  DP = _LANES

  x2 = x.astype(jnp.int32).reshape(B * _RPE, S // _RPE)
  b_pad = jnp.concatenate([b, jnp.zeros((_LANES - C,), b.dtype)])
  out_pad = _pool_stage(x2, table, W.T, b_pad, 1.0 / S, DP)
  return out_pad[:, :C]
